# baseline (speedup 1.0000x reference)
import dataclasses
import functools

import jax
import jax.numpy as jnp
from jax import lax
from jax.experimental import pallas as pl
from jax.experimental.pallas import tpu as pltpu
from jax.experimental.pallas import tpu_sc as plsc

N = 10000
E = 320000
H = 128
HH = 64
NC = 2
NS = 16
EPT = E // NS
CH = 48
NFULL = EPT // CH
TAIL = EPT - NFULL * CH
BE = 2000
F32 = jnp.float32


def _embed_body(x_ref, w_ref, b_ref, o_ref):
    o_ref[...] = jnp.dot(x_ref[...], w_ref[...],
                         preferred_element_type=F32) + b_ref[...]


def _embed(x, w, b):
    return pl.pallas_call(
        _embed_body,
        out_shape=jax.ShapeDtypeStruct((N, H), F32),
    )(x, w, b)


def _tables_body(h_ref, a_ref, b_ref, v_ref, oa_ref, os_ref):
    h = h_ref[...]
    ga = jnp.dot(h, a_ref[...], preferred_element_type=F32)
    gb = jnp.dot(h, b_ref[...], preferred_element_type=F32)
    gv = jnp.dot(h, v_ref[...], preferred_element_type=F32)
    oa_ref[...] = ga
    os_ref[0] = jnp.concatenate([gb[:, :HH], gv[:, :HH]], axis=1)
    os_ref[1] = jnp.concatenate([gb[:, HH:], gv[:, HH:]], axis=1)


def _tables(h, a, b, v):
    return pl.pallas_call(
        _tables_body,
        out_shape=[jax.ShapeDtypeStruct((N, H), F32),
                   jax.ShapeDtypeStruct((NC, N, H), F32)],
    )(h, a, b, v)


def _eec1_body(e_ref, we_ref, be_ref, c_ref, eb_ref, o_ref):
    ee = jnp.dot(e_ref[...], we_ref[...], preferred_element_type=F32) + be_ref[...]
    t = jnp.dot(ee, c_ref[...], preferred_element_type=F32) + eb_ref[...]
    o_ref[0] = t[:, :HH]
    o_ref[1] = t[:, HH:]


def _eec1(e, we, be, c, eb):
    return pl.pallas_call(
        _eec1_body,
        grid=(E // BE,),
        in_specs=[
            pl.BlockSpec((BE, 16), lambda i: (i, 0)),
            pl.BlockSpec((16, H), lambda i: (0, 0)),
            pl.BlockSpec((1, H), lambda i: (0, 0)),
            pl.BlockSpec((H, H), lambda i: (0, 0)),
            pl.BlockSpec((1, H), lambda i: (0, 0)),
        ],
        out_specs=pl.BlockSpec((NC, BE, HH), lambda i: (0, i, 0)),
        out_shape=jax.ShapeDtypeStruct((NC, E, HH), F32),
    )(e, we, be, c, eb)


def _eec2_body(e_ref, eh_ref, st_ref, we_ref, be_ref, bg_ref, bb_ref,
               c_ref, eb_ref, o_ref):
    ee1 = jnp.dot(e_ref[...], we_ref[...], preferred_element_type=F32) + be_ref[...]
    eh = jnp.concatenate([eh_ref[0], eh_ref[1]], axis=1)
    st = st_ref[...]
    ssum = jnp.concatenate([jnp.sum(st[:NS, 0, :], axis=0),
                            jnp.sum(st[NS:, 0, :], axis=0)]).reshape(1, H)
    ssq = jnp.concatenate([jnp.sum(st[:NS, 1, :], axis=0),
                           jnp.sum(st[NS:, 1, :], axis=0)]).reshape(1, H)
    mu = ssum / E
    var = ssq / E - mu * mu
    ehn = bg_ref[...] * (eh - mu) * lax.rsqrt(var + 1e-5) + bb_ref[...]
    eout1 = ee1 + jnp.maximum(ehn, 0.0)
    t = jnp.dot(eout1, c_ref[...], preferred_element_type=F32) + eb_ref[...]
    o_ref[0] = t[:, :HH]
    o_ref[1] = t[:, HH:]


def _eec2(e, ehat1, stats, we, be, bg, bb, c, eb):
    return pl.pallas_call(
        _eec2_body,
        grid=(E // BE,),
        in_specs=[
            pl.BlockSpec((BE, 16), lambda i: (i, 0)),
            pl.BlockSpec((NC, BE, HH), lambda i: (0, i, 0)),
            pl.BlockSpec((NC * NS, 2, HH), lambda i: (0, 0, 0)),
            pl.BlockSpec((16, H), lambda i: (0, 0)),
            pl.BlockSpec((1, H), lambda i: (0, 0)),
            pl.BlockSpec((1, H), lambda i: (0, 0)),
            pl.BlockSpec((1, H), lambda i: (0, 0)),
            pl.BlockSpec((H, H), lambda i: (0, 0)),
            pl.BlockSpec((1, H), lambda i: (0, 0)),
        ],
        out_specs=pl.BlockSpec((NC, BE, HH), lambda i: (0, i, 0)),
        out_shape=jax.ShapeDtypeStruct((NC, E, HH), F32),
    )(e, ehat1, stats, we, be, bg, bb, c, eb)


def _node_body(h_ref, nd_ref, u_ref, hb_ref, bg_ref, bb_ref, o_ref):
    h = h_ref[...]
    num = jnp.concatenate([nd_ref[0, :, :HH], nd_ref[1, :, :HH]], axis=1)
    den = jnp.concatenate([nd_ref[0, :, HH:], nd_ref[1, :, HH:]], axis=1) + 1e-6
    hh = jnp.dot(h, u_ref[...], preferred_element_type=F32) + num / den + hb_ref[...]
    mu = jnp.mean(hh, axis=0, keepdims=True)
    var = jnp.mean(hh * hh, axis=0, keepdims=True) - mu * mu
    hn = bg_ref[...] * (hh - mu) * lax.rsqrt(var + 1e-5) + bb_ref[...]
    o_ref[...] = h + jnp.maximum(hn, 0.0)


def _node_update(h, nd, u, hb, bg, bb):
    return pl.pallas_call(
        _node_body,
        out_shape=jax.ShapeDtypeStruct((N, H), F32),
    )(h, nd, u, hb, bg, bb)


def _head_body(h_ref, w1_ref, b1_ref, w2_ref, b2_ref, o_ref):
    z = jnp.dot(h_ref[...], w1_ref[...], preferred_element_type=F32) + b1_ref[...]
    z = jnp.maximum(z, 0.0)
    t = jnp.dot(z, w2_ref[...], preferred_element_type=F32) + b2_ref[...]
    o_ref[...] = -1.2 * jnp.tanh(t)


def _head(h, w1, b1, w2p, b2p):
    return pl.pallas_call(
        _head_body,
        out_shape=jax.ShapeDtypeStruct((N, H), F32),
    )(h, w1, b1, w2p, b2p)


_SIG_F = (-30.0, 30.0, -1.4426950408889634, 1.5 * 2 ** 23, 2.0, 1.0,
          1.0, 0.6931471805599453, 0.2402265069591007, 0.05550410866482158,
          0.009618129107628477, 0.0013333558146428443)
_SIG_I = (127 - 0x4B400000, 0x7EF311C3)
_NSIG = len(_SIG_F) + len(_SIG_I)


def _sig_consts():
    return tuple(jnp.full((16,), v, F32) for v in _SIG_F) + tuple(
        jnp.full((16,), v, jnp.int32) for v in _SIG_I)


def _sigmoid16(v, K):
    (lo, hi, nl2e, big, two, one, c0, c1, c2, c3, c4, c5, ebias, magic) = K
    vc = jnp.minimum(jnp.maximum(v, lo), hi)
    t = vc * nl2e
    u = t + big
    ni = plsc.bitcast(u, jnp.int32)
    sfac = plsc.bitcast(lax.shift_left(ni + ebias, 23), F32)
    f = t - (u - big)
    p2 = ((((c5 * f + c4) * f + c3) * f + c2) * f + c1) * f + c0
    x = one + p2 * sfac
    y = plsc.bitcast(magic - plsc.bitcast(x, jnp.int32), F32)
    y = y * (two - x * y)
    y = y * (two - x * y)
    return y

def _make_sc_edge(first):
    mesh = plsc.VectorSubcoreMesh(core_axis_name="c", subcore_axis_name="s")
    outs = [jax.ShapeDtypeStruct((NC * N, H), F32)]
    if first:
        outs.append(jax.ShapeDtypeStruct((NC * E // 2, H), F32))
        outs.append(jax.ShapeDtypeStruct((NC * NS, 2, HH), F32))

    def bufset():
        return [
            pltpu.VMEM((CH,), jnp.int32),
            pltpu.VMEM((CH,), jnp.int32),
            pltpu.VMEM((CH,), jnp.int32),
            pltpu.VMEM((CH // 2, H), F32),
            pltpu.VMEM((CH, H), F32),
            pltpu.VMEM((CH, H), F32),
            pltpu.VMEM((CH, H), F32),
            pltpu.VMEM((CH // 2, H), F32),
            pltpu.SemaphoreType.DMA,
            pltpu.SemaphoreType.DMA,
            pltpu.SemaphoreType.DMA,
            pltpu.SemaphoreType.DMA,
            pltpu.VMEM((CH,), jnp.int32),
        ]

    scratch = ([pltpu.VMEM_SHARED((N, H), F32)] + bufset() + bufset()
               + [pltpu.VMEM((2, HH), F32),
                  pltpu.VMEM((TAIL,), jnp.int32),
                  pltpu.VMEM((TAIL,), jnp.int32),
                  pltpu.VMEM((TAIL,), jnp.int32)])

    cp = pltpu.CompilerParams()
    if "needs_layout_passes" in pltpu.CompilerParams.__dataclass_fields__:
        cp = dataclasses.replace(cp, needs_layout_passes=False)

    @functools.partial(pl.kernel, out_type=outs, mesh=mesh,
                       scratch_types=scratch, compiler_params=cp)
    def k(eec, tga, tsrc, dst, src, *rest):
        if first:
            nd_out, ehat_out, stats_out = rest[0], rest[1], rest[2]
            scr = rest[3:]
        else:
            nd_out = rest[0]
            ehat_out = stats_out = None
            scr = rest[1:]
        acc = scr[0]
        S0 = scr[1:14]
        S1 = scr[14:27]
        bS = scr[27]
        dIt, sIt, sAt = scr[28], scr[29], scr[30]
        c = lax.axis_index("c")
        s = lax.axis_index("s")
        coff = c * N
        proff = c * (E // 2) + s * (EPT // 2)
        base0 = s * EPT
        zero16 = jnp.zeros((16,), F32)

        bC0 = S0[6]

        @pl.loop(0, CH)
        def _(i):
            for j in range(H // 16):
                bC0[i, pl.ds(j * 16, 16)] = zero16
        rows = N // NS - 1
        nz, rz = rows // CH, rows % CH
        for t in range(nz):
            pltpu.sync_copy(bC0.at[pl.ds(0, CH)],
                            acc.at[pl.ds(s * rows + t * CH, CH)])
        if rz:
            pltpu.sync_copy(bC0.at[pl.ds(0, rz)],
                            acc.at[pl.ds(s * rows + nz * CH, rz)])

        @pl.when(s == NS - 1)
        def _():
            pltpu.sync_copy(bC0.at[pl.ds(0, N - NS * rows)],
                            acc.at[pl.ds(NS * rows, N - NS * rows)])
        plsc.subcore_barrier()

        def issue_idx(kk, S):
            b = base0 + kk * CH
            pltpu.async_copy(dst.at[pl.ds(b, CH)], S[0], S[8])
            pltpu.async_copy(src.at[pl.ds(b, CH)], S[1], S[8])

        def wait_idx(S):
            pltpu.make_async_copy(dst.at[pl.ds(0, CH)], S[0], S[8]).wait()
            pltpu.make_async_copy(src.at[pl.ds(0, CH)], S[1], S[8]).wait()

        def adjust(S):
            for j in range(CH // 16):
                S[2][pl.ds(j * 16, 16)] = S[1][pl.ds(j * 16, 16)] + coff

        def issue_gather(kk, S):
            pltpu.async_copy(eec.at[pl.ds(proff + kk * (CH // 2), CH // 2)],
                             S[3], S[9])
            pltpu.async_copy(tga.at[S[0]], S[4], S[9])
            pltpu.async_copy(tsrc.at[S[2]], S[5], S[9])

        def wait_gather(S):
            pltpu.make_async_copy(eec.at[pl.ds(0, CH // 2)], S[3], S[9]).wait()
            pltpu.make_async_copy(tga.at[S[0]], S[4], S[9]).wait()
            pltpu.make_async_copy(tsrc.at[S[2]], S[5], S[9]).wait()

        def issue_scatter(S):
            pltpu.async_copy(S[6], acc.at[S[12]], S[10], add=True)

        def wait_scatter(S):
            pltpu.make_async_copy(S[6], acc.at[S[12]], S[10]).wait()

        def issue_ehat(kk, S):
            pltpu.async_copy(
                S[7], ehat_out.at[pl.ds(proff + kk * (CH // 2), CH // 2)],
                S[11])

        def wait_ehat(S):
            pltpu.make_async_copy(
                S[7], ehat_out.at[pl.ds(0, CH // 2)], S[11]).wait()

        def compute(S, carry, n2):
            bE, bA, bV, bC, bH = S[3], S[4], S[5], S[6], S[7]

            def body(i2, cr):
                st = list(cr[:-_NSIG])
                K = cr[-_NSIG:]
                for half in range(2):
                    r = 2 * i2 + half
                    for j in range(HH // 16):
                        colE = half * HH + j * 16
                        v = (bE[i2, pl.ds(colE, 16)]
                             + bA[r, pl.ds(c * HH + j * 16, 16)]
                             + bV[r, pl.ds(j * 16, 16)])
                        if first:
                            bH[i2, pl.ds(colE, 16)] = v
                            st[j] = st[j] + v
                            st[4 + j] = st[4 + j] + v * v
                        sg = _sigmoid16(v, K)
                        bC[r, pl.ds(HH + j * 16, 16)] = sg
                        bC[r, pl.ds(j * 16, 16)] = (
                            sg * bV[r, pl.ds(HH + j * 16, 16)])
                return tuple(st) + tuple(K)

            return lax.fori_loop(0, n2, body, carry)

        def step(kk, SP, SQ, k1, k2, do_sw, stats):
            wait_gather(SP)
            wait_idx(SQ)
            adjust(SQ)
            issue_gather(k1, SQ)
            if do_sw:
                wait_scatter(SP)
                if first:
                    wait_ehat(SP)
            for j in range(CH // 16):
                SP[12][pl.ds(j * 16, 16)] = SP[0][pl.ds(j * 16, 16)]
            issue_idx(k2, SP)
            stats = compute(SP, stats, CH // 2)
            issue_scatter(SP)
            if first:
                issue_ehat(kk, SP)
            return stats

        if first:
            stats = tuple(jnp.zeros((16,), F32) for _ in range(8))
        else:
            stats = ()
        stats = stats + _sig_consts()

        issue_idx(0, S0)
        wait_idx(S0)
        adjust(S0)
        issue_gather(0, S0)
        issue_idx(1, S1)
        stats = step(0, S0, S1, 1, 2, False, stats)
        stats = step(1, S1, S0, 2, 3, False, stats)

        def loop_body(m, stats):
            kk = 2 * m
            k2a = jnp.minimum(kk + 2, NFULL - 1)
            k3a = jnp.minimum(kk + 3, NFULL - 1)
            stats = step(kk, S0, S1, kk + 1, k2a, True, stats)
            stats = step(kk + 1, S1, S0, k2a, k3a, True, stats)
            return stats

        stats = lax.fori_loop(1, NFULL // 2, loop_body, stats)

        wait_gather(S0)
        wait_idx(S1)
        wait_scatter(S0)
        wait_scatter(S1)
        if first:
            wait_ehat(S0)
            wait_ehat(S1)

        tb = base0 + NFULL * CH
        tp = proff + NFULL * (CH // 2)
        cpa = pltpu.async_copy(dst.at[pl.ds(tb, TAIL)], dIt, S0[8])
        cpb = pltpu.async_copy(src.at[pl.ds(tb, TAIL)], sIt, S0[8])
        cpa.wait()
        cpb.wait()
        for j in range(TAIL // 16):
            sAt[pl.ds(j * 16, 16)] = sIt[pl.ds(j * 16, 16)] + coff
        ga = pltpu.async_copy(eec.at[pl.ds(tp, TAIL // 2)],
                              S0[3].at[pl.ds(0, TAIL // 2)], S0[9])
        gb = pltpu.async_copy(tga.at[dIt], S0[4].at[pl.ds(0, TAIL)], S0[9])
        gc = pltpu.async_copy(tsrc.at[sAt], S0[5].at[pl.ds(0, TAIL)], S0[9])
        ga.wait()
        gb.wait()
        gc.wait()
        stats = compute(S0, stats, TAIL // 2)
        pltpu.sync_copy(S0[6].at[pl.ds(0, TAIL)], acc.at[dIt], add=True)
        if first:
            pltpu.sync_copy(S0[7].at[pl.ds(0, TAIL // 2)],
                            ehat_out.at[pl.ds(tp, TAIL // 2)])
            for j in range(HH // 16):
                bS[0, pl.ds(j * 16, 16)] = stats[j]
                bS[1, pl.ds(j * 16, 16)] = stats[4 + j]

        plsc.subcore_barrier()
        pltpu.sync_copy(acc.at[pl.ds(s * rows, rows)],
                        nd_out.at[pl.ds(coff + s * rows, rows)])

        @pl.when(s == NS - 1)
        def _():
            pltpu.sync_copy(acc.at[pl.ds(NS * rows, N - NS * rows)],
                            nd_out.at[pl.ds(coff + NS * rows, N - NS * rows)])
        if first:
            pltpu.sync_copy(bS, stats_out.at[c * NS + s])

    return k


_sc_edge_first = _make_sc_edge(True)
_sc_edge_rest = _make_sc_edge(False)


def kernel(x, e, edge_index, params):
    src = edge_index[0]
    dst = edge_index[1]
    p1, p2 = params['layers'][0], params['layers'][1]
    r = lambda b: b.reshape(1, H)

    h0 = _embed(x, params['emb_h_w'], r(params['emb_h_b']))
    eec1 = _eec1(e, params['emb_e_w'], r(params['emb_e_b']),
                 p1['C'], r(p1['e_b']))
    ta1, ts1 = _tables(h0, p1['A'], p1['B'], p1['V'])
    nd1, ehat1, stats1 = _sc_edge_first(
        eec1.reshape(NC * E // 2, H), ta1, ts1.reshape(NC * N, H), dst, src)
    h1 = _node_update(h0, nd1.reshape(NC, N, H), p1['U'], r(p1['h_b']),
                      r(p1['bn_h_g']), r(p1['bn_h_b']))

    eec2 = _eec2(e, ehat1.reshape(NC, E, HH), stats1,
                 params['emb_e_w'], r(params['emb_e_b']),
                 r(p1['bn_e_g']), r(p1['bn_e_b']), p2['C'], r(p2['e_b']))
    ta2, ts2 = _tables(h1, p2['A'], p2['B'], p2['V'])
    [nd2] = _sc_edge_rest(
        eec2.reshape(NC * E // 2, H), ta2, ts2.reshape(NC * N, H), dst, src)
    h2 = _node_update(h1, nd2.reshape(NC, N, H), p2['U'], r(p2['h_b']),
                      r(p2['bn_h_g']), r(p2['bn_h_b']))

    w2p = jnp.pad(params['mlp_w2'], ((0, 0), (0, H - 2)))
    b2p = jnp.pad(params['mlp_b2'], (0, H - 2)).reshape(1, H)
    out = _head(h2, params['mlp_w1'], r(params['mlp_b1']), w2p, b2p)
    return out[:, :2]


# speedup vs baseline: 1.8234x; 1.8234x over previous
import dataclasses
import functools

import jax
import jax.numpy as jnp
from jax import lax
from jax.experimental import pallas as pl
from jax.experimental.pallas import tpu as pltpu
from jax.experimental.pallas import tpu_sc as plsc

N = 10000
E = 320000
H = 128
HH = 64
NC = 2
NS = 16
EPT = E // NS
CH = 48
NFULL = EPT // CH
TAIL = EPT - NFULL * CH
BE = 2000
F32 = jnp.float32


def _embed_body(x_ref, w_ref, b_ref, o_ref):
    o_ref[...] = jnp.dot(x_ref[...], w_ref[...],
                         preferred_element_type=F32) + b_ref[...]


def _embed(x, w, b):
    return pl.pallas_call(
        _embed_body,
        out_shape=jax.ShapeDtypeStruct((N, H), F32),
    )(x, w, b)


def _tables_body(h_ref, a_ref, b_ref, v_ref, oa_ref, os_ref):
    h = h_ref[...]
    ga = jnp.dot(h, a_ref[...], preferred_element_type=F32)
    gb = jnp.dot(h, b_ref[...], preferred_element_type=F32)
    gv = jnp.dot(h, v_ref[...], preferred_element_type=F32)
    oa_ref[...] = ga
    os_ref[0] = jnp.concatenate([gb[:, :HH], gv[:, :HH]], axis=1)
    os_ref[1] = jnp.concatenate([gb[:, HH:], gv[:, HH:]], axis=1)


def _tables(h, a, b, v):
    return pl.pallas_call(
        _tables_body,
        out_shape=[jax.ShapeDtypeStruct((N, H), F32),
                   jax.ShapeDtypeStruct((NC, N, H), F32)],
    )(h, a, b, v)


def _eec1_body(e_ref, we_ref, be_ref, c_ref, eb_ref, o_ref):
    ee = jnp.dot(e_ref[...], we_ref[...], preferred_element_type=F32) + be_ref[...]
    t = jnp.dot(ee, c_ref[...], preferred_element_type=F32) + eb_ref[...]
    o_ref[0] = t[:, :HH]
    o_ref[1] = t[:, HH:]


def _eec1(e, we, be, c, eb):
    return pl.pallas_call(
        _eec1_body,
        grid=(E // BE,),
        in_specs=[
            pl.BlockSpec((BE, 16), lambda i: (i, 0)),
            pl.BlockSpec((16, H), lambda i: (0, 0)),
            pl.BlockSpec((1, H), lambda i: (0, 0)),
            pl.BlockSpec((H, H), lambda i: (0, 0)),
            pl.BlockSpec((1, H), lambda i: (0, 0)),
        ],
        out_specs=pl.BlockSpec((NC, BE, HH), lambda i: (0, i, 0)),
        out_shape=jax.ShapeDtypeStruct((NC, E, HH), F32),
    )(e, we, be, c, eb)


def _eec2_body(e_ref, eh_ref, st_ref, we_ref, be_ref, bg_ref, bb_ref,
               c_ref, eb_ref, o_ref):
    ee1 = jnp.dot(e_ref[...], we_ref[...], preferred_element_type=F32) + be_ref[...]
    eh = jnp.concatenate([eh_ref[0], eh_ref[1]], axis=1)
    st = st_ref[...]
    ssum = jnp.concatenate([jnp.sum(st[:NS, 0, :], axis=0),
                            jnp.sum(st[NS:, 0, :], axis=0)]).reshape(1, H)
    ssq = jnp.concatenate([jnp.sum(st[:NS, 1, :], axis=0),
                           jnp.sum(st[NS:, 1, :], axis=0)]).reshape(1, H)
    mu = ssum / E
    var = ssq / E - mu * mu
    ehn = bg_ref[...] * (eh - mu) * lax.rsqrt(var + 1e-5) + bb_ref[...]
    eout1 = ee1 + jnp.maximum(ehn, 0.0)
    t = jnp.dot(eout1, c_ref[...], preferred_element_type=F32) + eb_ref[...]
    o_ref[0] = t[:, :HH]
    o_ref[1] = t[:, HH:]


def _eec2(e, ehat1, stats, we, be, bg, bb, c, eb):
    return pl.pallas_call(
        _eec2_body,
        grid=(E // BE,),
        in_specs=[
            pl.BlockSpec((BE, 16), lambda i: (i, 0)),
            pl.BlockSpec((NC, BE, HH), lambda i: (0, i, 0)),
            pl.BlockSpec((NC * NS, 2, HH), lambda i: (0, 0, 0)),
            pl.BlockSpec((16, H), lambda i: (0, 0)),
            pl.BlockSpec((1, H), lambda i: (0, 0)),
            pl.BlockSpec((1, H), lambda i: (0, 0)),
            pl.BlockSpec((1, H), lambda i: (0, 0)),
            pl.BlockSpec((H, H), lambda i: (0, 0)),
            pl.BlockSpec((1, H), lambda i: (0, 0)),
        ],
        out_specs=pl.BlockSpec((NC, BE, HH), lambda i: (0, i, 0)),
        out_shape=jax.ShapeDtypeStruct((NC, E, HH), F32),
    )(e, ehat1, stats, we, be, bg, bb, c, eb)


def _node_body(h_ref, nd_ref, u_ref, hb_ref, bg_ref, bb_ref, o_ref):
    h = h_ref[...]
    num = jnp.concatenate([nd_ref[0, :, :HH], nd_ref[1, :, :HH]], axis=1)
    den = jnp.concatenate([nd_ref[0, :, HH:], nd_ref[1, :, HH:]], axis=1) + 1e-6
    hh = jnp.dot(h, u_ref[...], preferred_element_type=F32) + num / den + hb_ref[...]
    mu = jnp.mean(hh, axis=0, keepdims=True)
    var = jnp.mean(hh * hh, axis=0, keepdims=True) - mu * mu
    hn = bg_ref[...] * (hh - mu) * lax.rsqrt(var + 1e-5) + bb_ref[...]
    o_ref[...] = h + jnp.maximum(hn, 0.0)


def _node_update(h, nd, u, hb, bg, bb):
    return pl.pallas_call(
        _node_body,
        out_shape=jax.ShapeDtypeStruct((N, H), F32),
    )(h, nd, u, hb, bg, bb)


def _head_body(h_ref, w1_ref, b1_ref, w2_ref, b2_ref, o_ref):
    z = jnp.dot(h_ref[...], w1_ref[...], preferred_element_type=F32) + b1_ref[...]
    z = jnp.maximum(z, 0.0)
    t = jnp.dot(z, w2_ref[...], preferred_element_type=F32) + b2_ref[...]
    o_ref[...] = -1.2 * jnp.tanh(t)


def _head(h, w1, b1, w2p, b2p):
    return pl.pallas_call(
        _head_body,
        out_shape=jax.ShapeDtypeStruct((N, H), F32),
    )(h, w1, b1, w2p, b2p)


_SIG_F = (-30.0, 30.0, -1.4426950408889634, 1.5 * 2 ** 23, 2.0, 1.0,
          1.0, 0.6931471805599453, 0.2402265069591007, 0.05550410866482158,
          0.009618129107628477, 0.0013333558146428443)
_SIG_I = (127 - 0x4B400000, 0x7EF311C3)
_NSIG = len(_SIG_F) + len(_SIG_I)


def _sig_consts():
    return tuple(jnp.full((16,), v, F32) for v in _SIG_F) + tuple(
        jnp.full((16,), v, jnp.int32) for v in _SIG_I)


def _sigmoid16(v, K):
    (lo, hi, nl2e, big, two, one, c0, c1, c2, c3, c4, c5, ebias, magic) = K
    vc = jnp.minimum(jnp.maximum(v, lo), hi)
    t = vc * nl2e
    u = t + big
    ni = plsc.bitcast(u, jnp.int32)
    sfac = plsc.bitcast(lax.shift_left(ni + ebias, 23), F32)
    f = t - (u - big)
    p2 = ((((c5 * f + c4) * f + c3) * f + c2) * f + c1) * f + c0
    x = one + p2 * sfac
    y = plsc.bitcast(magic - plsc.bitcast(x, jnp.int32), F32)
    y = y * (two - x * y)
    y = y * (two - x * y)
    return y

def _make_sc_edge(first):
    mesh = plsc.VectorSubcoreMesh(core_axis_name="c", subcore_axis_name="s")
    outs = [jax.ShapeDtypeStruct((NC * N, H), F32)]
    if first:
        outs.append(jax.ShapeDtypeStruct((NC * E // 2, H), F32))
        outs.append(jax.ShapeDtypeStruct((NC * NS, 2, HH), F32))

    def bufset():
        return [
            pltpu.VMEM((CH,), jnp.int32),
            pltpu.VMEM((CH,), jnp.int32),
            pltpu.VMEM((CH,), jnp.int32),
            pltpu.VMEM((CH // 2, H), F32),
            pltpu.VMEM((CH, H), F32),
            pltpu.VMEM((CH, H), F32),
            pltpu.VMEM((CH, H), F32),
            pltpu.VMEM((CH // 2, H), F32),
            pltpu.SemaphoreType.DMA,
            pltpu.SemaphoreType.DMA,
            pltpu.SemaphoreType.DMA,
            pltpu.SemaphoreType.DMA,
            pltpu.VMEM((CH,), jnp.int32),
        ]

    scratch = ([pltpu.VMEM_SHARED((N, H), F32)] + bufset() + bufset()
               + [pltpu.VMEM((2, HH), F32),
                  pltpu.VMEM((TAIL,), jnp.int32),
                  pltpu.VMEM((TAIL,), jnp.int32),
                  pltpu.VMEM((TAIL,), jnp.int32)])

    cp = pltpu.CompilerParams()
    if "needs_layout_passes" in pltpu.CompilerParams.__dataclass_fields__:
        cp = dataclasses.replace(cp, needs_layout_passes=False)

    @functools.partial(pl.kernel, out_type=outs, mesh=mesh,
                       scratch_types=scratch, compiler_params=cp)
    def k(eec, tga, tsrc, dst, src, *rest):
        if first:
            nd_out, ehat_out, stats_out = rest[0], rest[1], rest[2]
            scr = rest[3:]
        else:
            nd_out = rest[0]
            ehat_out = stats_out = None
            scr = rest[1:]
        acc = scr[0]
        S0 = scr[1:14]
        S1 = scr[14:27]
        bS = scr[27]
        dIt, sIt, sAt = scr[28], scr[29], scr[30]
        c = lax.axis_index("c")
        s = lax.axis_index("s")
        coff = c * N
        proff = c * (E // 2) + s * (EPT // 2)
        base0 = s * EPT
        zero16 = jnp.zeros((16,), F32)

        bC0 = S0[6]

        @pl.loop(0, CH)
        def _(i):
            for j in range(H // 16):
                bC0[i, pl.ds(j * 16, 16)] = zero16
        rows = N // NS - 1
        nz, rz = rows // CH, rows % CH
        for t in range(nz):
            pltpu.sync_copy(bC0.at[pl.ds(0, CH)],
                            acc.at[pl.ds(s * rows + t * CH, CH)])
        if rz:
            pltpu.sync_copy(bC0.at[pl.ds(0, rz)],
                            acc.at[pl.ds(s * rows + nz * CH, rz)])

        @pl.when(s == NS - 1)
        def _():
            pltpu.sync_copy(bC0.at[pl.ds(0, N - NS * rows)],
                            acc.at[pl.ds(NS * rows, N - NS * rows)])
        plsc.subcore_barrier()

        def issue_idx(kk, S):
            b = base0 + kk * CH
            pltpu.async_copy(dst.at[pl.ds(b, CH)], S[0], S[8])
            pltpu.async_copy(src.at[pl.ds(b, CH)], S[1], S[8])

        def wait_idx(S):
            pltpu.make_async_copy(dst.at[pl.ds(0, CH)], S[0], S[8]).wait()
            pltpu.make_async_copy(src.at[pl.ds(0, CH)], S[1], S[8]).wait()

        def adjust(S):
            for j in range(CH // 16):
                S[2][pl.ds(j * 16, 16)] = S[1][pl.ds(j * 16, 16)] + coff

        def issue_gather(kk, S):
            pltpu.async_copy(eec.at[pl.ds(proff + kk * (CH // 2), CH // 2)],
                             S[3], S[9])
            pltpu.async_copy(tga.at[S[0]], S[4], S[9])
            pltpu.async_copy(tsrc.at[S[2]], S[5], S[9])

        def wait_gather(S):
            pltpu.make_async_copy(eec.at[pl.ds(0, CH // 2)], S[3], S[9]).wait()
            pltpu.make_async_copy(tga.at[S[0]], S[4], S[9]).wait()
            pltpu.make_async_copy(tsrc.at[S[2]], S[5], S[9]).wait()

        def issue_scatter(S):
            pltpu.async_copy(S[6], acc.at[S[12]], S[10], add=True)

        def wait_scatter(S):
            pltpu.make_async_copy(S[6], acc.at[S[12]], S[10]).wait()

        def issue_ehat(kk, S):
            pltpu.async_copy(
                S[7], ehat_out.at[pl.ds(proff + kk * (CH // 2), CH // 2)],
                S[11])

        def wait_ehat(S):
            pltpu.make_async_copy(
                S[7], ehat_out.at[pl.ds(0, CH // 2)], S[11]).wait()

        def compute(S, carry, n2):
            bE, bA, bV, bC, bH = S[3], S[4], S[5], S[6], S[7]

            def body(i2, cr):
                st = list(cr)
                for half in range(2):
                    r = 2 * i2 + half
                    for j in range(HH // 16):
                        colE = half * HH + j * 16
                        v = (bE[i2, pl.ds(colE, 16)]
                             + bA[r, pl.ds(c * HH + j * 16, 16)]
                             + bV[r, pl.ds(j * 16, 16)])
                        if first:
                            bH[i2, pl.ds(colE, 16)] = v
                            st[j] = st[j] + v
                            st[4 + j] = st[4 + j] + v * v
                        sg = 1.0 / (1.0 + jnp.exp(-v))
                        bC[r, pl.ds(HH + j * 16, 16)] = sg
                        bC[r, pl.ds(j * 16, 16)] = (
                            sg * bV[r, pl.ds(HH + j * 16, 16)])
                if first:
                    return tuple(st)

            if first:
                return lax.fori_loop(0, n2, body, carry)
            plsc.parallel_loop(0, n2)(lambda i2: body(i2, ()))
            return carry

        def step(kk, SP, SQ, k1, k2, do_sw, stats):
            wait_gather(SP)
            wait_idx(SQ)
            adjust(SQ)
            issue_gather(k1, SQ)
            if do_sw:
                wait_scatter(SP)
                if first:
                    wait_ehat(SP)
            for j in range(CH // 16):
                SP[12][pl.ds(j * 16, 16)] = SP[0][pl.ds(j * 16, 16)]
            issue_idx(k2, SP)
            stats = compute(SP, stats, CH // 2)
            issue_scatter(SP)
            if first:
                issue_ehat(kk, SP)
            return stats

        if first:
            stats = tuple(jnp.zeros((16,), F32) for _ in range(8))
        else:
            stats = ()

        issue_idx(0, S0)
        wait_idx(S0)
        adjust(S0)
        issue_gather(0, S0)
        issue_idx(1, S1)
        stats = step(0, S0, S1, 1, 2, False, stats)
        stats = step(1, S1, S0, 2, 3, False, stats)

        def loop_body(m, stats):
            kk = 2 * m
            k2a = jnp.minimum(kk + 2, NFULL - 1)
            k3a = jnp.minimum(kk + 3, NFULL - 1)
            stats = step(kk, S0, S1, kk + 1, k2a, True, stats)
            stats = step(kk + 1, S1, S0, k2a, k3a, True, stats)
            return stats

        stats = lax.fori_loop(1, NFULL // 2, loop_body, stats)

        wait_gather(S0)
        wait_idx(S1)
        wait_scatter(S0)
        wait_scatter(S1)
        if first:
            wait_ehat(S0)
            wait_ehat(S1)

        tb = base0 + NFULL * CH
        tp = proff + NFULL * (CH // 2)
        cpa = pltpu.async_copy(dst.at[pl.ds(tb, TAIL)], dIt, S0[8])
        cpb = pltpu.async_copy(src.at[pl.ds(tb, TAIL)], sIt, S0[8])
        cpa.wait()
        cpb.wait()
        for j in range(TAIL // 16):
            sAt[pl.ds(j * 16, 16)] = sIt[pl.ds(j * 16, 16)] + coff
        ga = pltpu.async_copy(eec.at[pl.ds(tp, TAIL // 2)],
                              S0[3].at[pl.ds(0, TAIL // 2)], S0[9])
        gb = pltpu.async_copy(tga.at[dIt], S0[4].at[pl.ds(0, TAIL)], S0[9])
        gc = pltpu.async_copy(tsrc.at[sAt], S0[5].at[pl.ds(0, TAIL)], S0[9])
        ga.wait()
        gb.wait()
        gc.wait()
        stats = compute(S0, stats, TAIL // 2)
        pltpu.sync_copy(S0[6].at[pl.ds(0, TAIL)], acc.at[dIt], add=True)
        if first:
            pltpu.sync_copy(S0[7].at[pl.ds(0, TAIL // 2)],
                            ehat_out.at[pl.ds(tp, TAIL // 2)])
            for j in range(HH // 16):
                bS[0, pl.ds(j * 16, 16)] = stats[j]
                bS[1, pl.ds(j * 16, 16)] = stats[4 + j]

        plsc.subcore_barrier()
        pltpu.sync_copy(acc.at[pl.ds(s * rows, rows)],
                        nd_out.at[pl.ds(coff + s * rows, rows)])

        @pl.when(s == NS - 1)
        def _():
            pltpu.sync_copy(acc.at[pl.ds(NS * rows, N - NS * rows)],
                            nd_out.at[pl.ds(coff + NS * rows, N - NS * rows)])
        if first:
            pltpu.sync_copy(bS, stats_out.at[c * NS + s])

    return k


_sc_edge_first = _make_sc_edge(True)
_sc_edge_rest = _make_sc_edge(False)


def kernel(x, e, edge_index, params):
    src = edge_index[0]
    dst = edge_index[1]
    p1, p2 = params['layers'][0], params['layers'][1]
    r = lambda b: b.reshape(1, H)

    h0 = _embed(x, params['emb_h_w'], r(params['emb_h_b']))
    eec1 = _eec1(e, params['emb_e_w'], r(params['emb_e_b']),
                 p1['C'], r(p1['e_b']))
    ta1, ts1 = _tables(h0, p1['A'], p1['B'], p1['V'])
    nd1, ehat1, stats1 = _sc_edge_first(
        eec1.reshape(NC * E // 2, H), ta1, ts1.reshape(NC * N, H), dst, src)
    h1 = _node_update(h0, nd1.reshape(NC, N, H), p1['U'], r(p1['h_b']),
                      r(p1['bn_h_g']), r(p1['bn_h_b']))

    eec2 = _eec2(e, ehat1.reshape(NC, E, HH), stats1,
                 params['emb_e_w'], r(params['emb_e_b']),
                 r(p1['bn_e_g']), r(p1['bn_e_b']), p2['C'], r(p2['e_b']))
    ta2, ts2 = _tables(h1, p2['A'], p2['B'], p2['V'])
    [nd2] = _sc_edge_rest(
        eec2.reshape(NC * E // 2, H), ta2, ts2.reshape(NC * N, H), dst, src)
    h2 = _node_update(h1, nd2.reshape(NC, N, H), p2['U'], r(p2['h_b']),
                      r(p2['bn_h_g']), r(p2['bn_h_b']))

    w2p = jnp.pad(params['mlp_w2'], ((0, 0), (0, H - 2)))
    b2p = jnp.pad(params['mlp_b2'], (0, H - 2)).reshape(1, H)
    out = _head(h2, params['mlp_w1'], r(params['mlp_b1']), w2p, b2p)
    return out[:, :2]


# speedup vs baseline: 2.4872x; 1.3640x over previous
import dataclasses
import functools

import jax
import jax.numpy as jnp
from jax import lax
from jax.experimental import pallas as pl
from jax.experimental.pallas import tpu as pltpu
from jax.experimental.pallas import tpu_sc as plsc

N = 10000
E = 320000
H = 128
HH = 64
NC = 2
NS = 16
EPT = E // NS
CH = 48
NFULL = EPT // CH
TAIL = EPT - NFULL * CH
BE = 2000
F32 = jnp.float32


def _embed_body(x_ref, w_ref, b_ref, o_ref):
    o_ref[...] = jnp.dot(x_ref[...], w_ref[...],
                         preferred_element_type=F32) + b_ref[...]


def _embed(x, w, b):
    return pl.pallas_call(
        _embed_body,
        out_shape=jax.ShapeDtypeStruct((N, H), F32),
    )(x, w, b)


def _tables_body(h_ref, a_ref, b_ref, v_ref, oa_ref, os_ref):
    h = h_ref[...]
    ga = jnp.dot(h, a_ref[...], preferred_element_type=F32)
    gb = jnp.dot(h, b_ref[...], preferred_element_type=F32)
    gv = jnp.dot(h, v_ref[...], preferred_element_type=F32)
    oa_ref[...] = ga
    os_ref[0] = jnp.concatenate([gb[:, :HH], gv[:, :HH]], axis=1)
    os_ref[1] = jnp.concatenate([gb[:, HH:], gv[:, HH:]], axis=1)


def _tables(h, a, b, v):
    return pl.pallas_call(
        _tables_body,
        out_shape=[jax.ShapeDtypeStruct((N, H), F32),
                   jax.ShapeDtypeStruct((NC, N, H), F32)],
    )(h, a, b, v)


def _eec1_body(e_ref, we_ref, be_ref, c_ref, eb_ref, o_ref):
    ee = jnp.dot(e_ref[...], we_ref[...], preferred_element_type=F32) + be_ref[...]
    t = jnp.dot(ee, c_ref[...], preferred_element_type=F32) + eb_ref[...]
    o_ref[0] = t[:, :HH]
    o_ref[1] = t[:, HH:]


def _eec1(e, we, be, c, eb):
    return pl.pallas_call(
        _eec1_body,
        grid=(E // BE,),
        in_specs=[
            pl.BlockSpec((BE, 16), lambda i: (i, 0)),
            pl.BlockSpec((16, H), lambda i: (0, 0)),
            pl.BlockSpec((1, H), lambda i: (0, 0)),
            pl.BlockSpec((H, H), lambda i: (0, 0)),
            pl.BlockSpec((1, H), lambda i: (0, 0)),
        ],
        out_specs=pl.BlockSpec((NC, BE, HH), lambda i: (0, i, 0)),
        out_shape=jax.ShapeDtypeStruct((NC, E, HH), F32),
    )(e, we, be, c, eb)


def _estats_body(eh_ref, o_ref):
    i = pl.program_id(0)
    eh = jnp.concatenate([eh_ref[0], eh_ref[1]], axis=1)

    @pl.when(i == 0)
    def _():
        o_ref[...] = jnp.zeros((2, H), F32)

    o_ref[0:1, :] += jnp.sum(eh, axis=0, keepdims=True)
    o_ref[1:2, :] += jnp.sum(eh * eh, axis=0, keepdims=True)


def _estats(ehat1):
    return pl.pallas_call(
        _estats_body,
        grid=(E // BE,),
        in_specs=[pl.BlockSpec((NC, BE, HH), lambda i: (0, i, 0))],
        out_specs=pl.BlockSpec((2, H), lambda i: (0, 0)),
        out_shape=jax.ShapeDtypeStruct((2, H), F32),
    )(ehat1)


def _eec2_body(e_ref, eh_ref, st_ref, we_ref, be_ref, bg_ref, bb_ref,
               c_ref, eb_ref, o_ref):
    ee1 = jnp.dot(e_ref[...], we_ref[...], preferred_element_type=F32) + be_ref[...]
    eh = jnp.concatenate([eh_ref[0], eh_ref[1]], axis=1)
    ssum = st_ref[0:1, :]
    ssq = st_ref[1:2, :]
    mu = ssum / E
    var = ssq / E - mu * mu
    ehn = bg_ref[...] * (eh - mu) * lax.rsqrt(var + 1e-5) + bb_ref[...]
    eout1 = ee1 + jnp.maximum(ehn, 0.0)
    t = jnp.dot(eout1, c_ref[...], preferred_element_type=F32) + eb_ref[...]
    o_ref[0] = t[:, :HH]
    o_ref[1] = t[:, HH:]


def _eec2(e, ehat1, stats, we, be, bg, bb, c, eb):
    return pl.pallas_call(
        _eec2_body,
        grid=(E // BE,),
        in_specs=[
            pl.BlockSpec((BE, 16), lambda i: (i, 0)),
            pl.BlockSpec((NC, BE, HH), lambda i: (0, i, 0)),
            pl.BlockSpec((2, H), lambda i: (0, 0)),
            pl.BlockSpec((16, H), lambda i: (0, 0)),
            pl.BlockSpec((1, H), lambda i: (0, 0)),
            pl.BlockSpec((1, H), lambda i: (0, 0)),
            pl.BlockSpec((1, H), lambda i: (0, 0)),
            pl.BlockSpec((H, H), lambda i: (0, 0)),
            pl.BlockSpec((1, H), lambda i: (0, 0)),
        ],
        out_specs=pl.BlockSpec((NC, BE, HH), lambda i: (0, i, 0)),
        out_shape=jax.ShapeDtypeStruct((NC, E, HH), F32),
    )(e, ehat1, stats, we, be, bg, bb, c, eb)


def _node_body(h_ref, nd_ref, u_ref, hb_ref, bg_ref, bb_ref, o_ref):
    h = h_ref[...]
    num = jnp.concatenate([nd_ref[0, :, :HH], nd_ref[1, :, :HH]], axis=1)
    den = jnp.concatenate([nd_ref[0, :, HH:], nd_ref[1, :, HH:]], axis=1) + 1e-6
    hh = jnp.dot(h, u_ref[...], preferred_element_type=F32) + num / den + hb_ref[...]
    mu = jnp.mean(hh, axis=0, keepdims=True)
    var = jnp.mean(hh * hh, axis=0, keepdims=True) - mu * mu
    hn = bg_ref[...] * (hh - mu) * lax.rsqrt(var + 1e-5) + bb_ref[...]
    o_ref[...] = h + jnp.maximum(hn, 0.0)


def _node_update(h, nd, u, hb, bg, bb):
    return pl.pallas_call(
        _node_body,
        out_shape=jax.ShapeDtypeStruct((N, H), F32),
    )(h, nd, u, hb, bg, bb)


def _head_body(h_ref, w1_ref, b1_ref, w2_ref, b2_ref, o_ref):
    z = jnp.dot(h_ref[...], w1_ref[...], preferred_element_type=F32) + b1_ref[...]
    z = jnp.maximum(z, 0.0)
    t = jnp.dot(z, w2_ref[...], preferred_element_type=F32) + b2_ref[...]
    o_ref[...] = -1.2 * jnp.tanh(t)


def _head(h, w1, b1, w2p, b2p):
    return pl.pallas_call(
        _head_body,
        out_shape=jax.ShapeDtypeStruct((N, H), F32),
    )(h, w1, b1, w2p, b2p)


def _make_sc_edge(first):
    mesh = plsc.VectorSubcoreMesh(core_axis_name="c", subcore_axis_name="s")
    outs = [jax.ShapeDtypeStruct((NC * N, H), F32)]
    if first:
        outs.append(jax.ShapeDtypeStruct((NC * E // 2, H), F32))

    def bufset():
        return [
            pltpu.VMEM((CH,), jnp.int32),
            pltpu.VMEM((CH,), jnp.int32),
            pltpu.VMEM((CH,), jnp.int32),
            pltpu.VMEM((CH // 2, H), F32),
            pltpu.VMEM((CH, H), F32),
            pltpu.VMEM((CH, H), F32),
            pltpu.VMEM((CH, H), F32),
            pltpu.VMEM((CH // 2, H), F32),
            pltpu.SemaphoreType.DMA,
            pltpu.SemaphoreType.DMA,
            pltpu.SemaphoreType.DMA,
            pltpu.SemaphoreType.DMA,
            pltpu.VMEM((CH,), jnp.int32),
        ]

    scratch = ([pltpu.VMEM_SHARED((N, H), F32)] + bufset() + bufset()
               + [pltpu.VMEM((TAIL,), jnp.int32),
                  pltpu.VMEM((TAIL,), jnp.int32),
                  pltpu.VMEM((TAIL,), jnp.int32)])

    cp = pltpu.CompilerParams()
    if "needs_layout_passes" in pltpu.CompilerParams.__dataclass_fields__:
        cp = dataclasses.replace(cp, needs_layout_passes=False)

    @functools.partial(pl.kernel, out_type=outs, mesh=mesh,
                       scratch_types=scratch, compiler_params=cp)
    def k(eec, tga, tsrc, dst, src, *rest):
        if first:
            nd_out, ehat_out = rest[0], rest[1]
            scr = rest[2:]
        else:
            nd_out = rest[0]
            ehat_out = None
            scr = rest[1:]
        acc = scr[0]
        S0 = scr[1:14]
        S1 = scr[14:27]
        dIt, sIt, sAt = scr[27], scr[28], scr[29]
        c = lax.axis_index("c")
        s = lax.axis_index("s")
        coff = c * N
        proff = c * (E // 2) + s * (EPT // 2)
        base0 = s * EPT
        zero16 = jnp.zeros((16,), F32)

        bC0 = S0[6]

        @pl.loop(0, CH)
        def _(i):
            for j in range(H // 16):
                bC0[i, pl.ds(j * 16, 16)] = zero16
        rows = N // NS - 1
        nz, rz = rows // CH, rows % CH
        for t in range(nz):
            pltpu.sync_copy(bC0.at[pl.ds(0, CH)],
                            acc.at[pl.ds(s * rows + t * CH, CH)])
        if rz:
            pltpu.sync_copy(bC0.at[pl.ds(0, rz)],
                            acc.at[pl.ds(s * rows + nz * CH, rz)])

        @pl.when(s == NS - 1)
        def _():
            pltpu.sync_copy(bC0.at[pl.ds(0, N - NS * rows)],
                            acc.at[pl.ds(NS * rows, N - NS * rows)])
        plsc.subcore_barrier()

        def issue_idx(kk, S):
            b = base0 + kk * CH
            pltpu.async_copy(dst.at[pl.ds(b, CH)], S[0], S[8])
            pltpu.async_copy(src.at[pl.ds(b, CH)], S[1], S[8])

        def wait_idx(S):
            pltpu.make_async_copy(dst.at[pl.ds(0, CH)], S[0], S[8]).wait()
            pltpu.make_async_copy(src.at[pl.ds(0, CH)], S[1], S[8]).wait()

        def adjust(S):
            for j in range(CH // 16):
                S[2][pl.ds(j * 16, 16)] = S[1][pl.ds(j * 16, 16)] + coff

        def issue_gather(kk, S):
            pltpu.async_copy(eec.at[pl.ds(proff + kk * (CH // 2), CH // 2)],
                             S[3], S[9])
            pltpu.async_copy(tga.at[S[0]], S[4], S[9])
            pltpu.async_copy(tsrc.at[S[2]], S[5], S[9])

        def wait_gather(S):
            pltpu.make_async_copy(eec.at[pl.ds(0, CH // 2)], S[3], S[9]).wait()
            pltpu.make_async_copy(tga.at[S[0]], S[4], S[9]).wait()
            pltpu.make_async_copy(tsrc.at[S[2]], S[5], S[9]).wait()

        def issue_scatter(S):
            pltpu.async_copy(S[6], acc.at[S[12]], S[10], add=True)

        def wait_scatter(S):
            pltpu.make_async_copy(S[6], acc.at[S[12]], S[10]).wait()

        def issue_ehat(kk, S):
            pltpu.async_copy(
                S[7], ehat_out.at[pl.ds(proff + kk * (CH // 2), CH // 2)],
                S[11])

        def wait_ehat(S):
            pltpu.make_async_copy(
                S[7], ehat_out.at[pl.ds(0, CH // 2)], S[11]).wait()

        def compute(S, n2):
            bE, bA, bV, bC, bH = S[3], S[4], S[5], S[6], S[7]

            @plsc.parallel_loop(0, n2)
            def _(i2):
                for half in range(2):
                    r = 2 * i2 + half
                    for j in range(HH // 16):
                        colE = half * HH + j * 16
                        v = (bE[i2, pl.ds(colE, 16)]
                             + bA[r, pl.ds(c * HH + j * 16, 16)]
                             + bV[r, pl.ds(j * 16, 16)])
                        if first:
                            bH[i2, pl.ds(colE, 16)] = v
                        sg = 1.0 / (1.0 + jnp.exp(-v))
                        bC[r, pl.ds(HH + j * 16, 16)] = sg
                        bC[r, pl.ds(j * 16, 16)] = (
                            sg * bV[r, pl.ds(HH + j * 16, 16)])

        def step(kk, SP, SQ, k1, k2, do_sw):
            wait_gather(SP)
            wait_idx(SQ)
            adjust(SQ)
            issue_gather(k1, SQ)
            if do_sw:
                wait_scatter(SP)
                if first:
                    wait_ehat(SP)
            for j in range(CH // 16):
                SP[12][pl.ds(j * 16, 16)] = SP[0][pl.ds(j * 16, 16)]
            issue_idx(k2, SP)
            compute(SP, CH // 2)
            issue_scatter(SP)
            if first:
                issue_ehat(kk, SP)

        issue_idx(0, S0)
        wait_idx(S0)
        adjust(S0)
        issue_gather(0, S0)
        issue_idx(1, S1)
        step(0, S0, S1, 1, 2, False)
        step(1, S1, S0, 2, 3, False)

        @pl.loop(1, NFULL // 2)
        def _(m):
            kk = 2 * m
            k2a = jnp.minimum(kk + 2, NFULL - 1)
            k3a = jnp.minimum(kk + 3, NFULL - 1)
            step(kk, S0, S1, kk + 1, k2a, True)
            step(kk + 1, S1, S0, k2a, k3a, True)

        wait_gather(S0)
        wait_idx(S1)
        wait_scatter(S0)
        wait_scatter(S1)
        if first:
            wait_ehat(S0)
            wait_ehat(S1)

        tb = base0 + NFULL * CH
        tp = proff + NFULL * (CH // 2)
        cpa = pltpu.async_copy(dst.at[pl.ds(tb, TAIL)], dIt, S0[8])
        cpb = pltpu.async_copy(src.at[pl.ds(tb, TAIL)], sIt, S0[8])
        cpa.wait()
        cpb.wait()
        for j in range(TAIL // 16):
            sAt[pl.ds(j * 16, 16)] = sIt[pl.ds(j * 16, 16)] + coff
        ga = pltpu.async_copy(eec.at[pl.ds(tp, TAIL // 2)],
                              S0[3].at[pl.ds(0, TAIL // 2)], S0[9])
        gb = pltpu.async_copy(tga.at[dIt], S0[4].at[pl.ds(0, TAIL)], S0[9])
        gc = pltpu.async_copy(tsrc.at[sAt], S0[5].at[pl.ds(0, TAIL)], S0[9])
        ga.wait()
        gb.wait()
        gc.wait()
        compute(S0, TAIL // 2)
        pltpu.sync_copy(S0[6].at[pl.ds(0, TAIL)], acc.at[dIt], add=True)
        if first:
            pltpu.sync_copy(S0[7].at[pl.ds(0, TAIL // 2)],
                            ehat_out.at[pl.ds(tp, TAIL // 2)])

        plsc.subcore_barrier()
        pltpu.sync_copy(acc.at[pl.ds(s * rows, rows)],
                        nd_out.at[pl.ds(coff + s * rows, rows)])

        @pl.when(s == NS - 1)
        def _():
            pltpu.sync_copy(acc.at[pl.ds(NS * rows, N - NS * rows)],
                            nd_out.at[pl.ds(coff + NS * rows, N - NS * rows)])

    return k


_sc_edge_first = _make_sc_edge(True)
_sc_edge_rest = _make_sc_edge(False)


def kernel(x, e, edge_index, params):
    src = edge_index[0]
    dst = edge_index[1]
    p1, p2 = params['layers'][0], params['layers'][1]
    r = lambda b: b.reshape(1, H)

    h0 = _embed(x, params['emb_h_w'], r(params['emb_h_b']))
    eec1 = _eec1(e, params['emb_e_w'], r(params['emb_e_b']),
                 p1['C'], r(p1['e_b']))
    ta1, ts1 = _tables(h0, p1['A'], p1['B'], p1['V'])
    nd1, ehat1 = _sc_edge_first(
        eec1.reshape(NC * E // 2, H), ta1, ts1.reshape(NC * N, H), dst, src)
    h1 = _node_update(h0, nd1.reshape(NC, N, H), p1['U'], r(p1['h_b']),
                      r(p1['bn_h_g']), r(p1['bn_h_b']))
    stats1 = _estats(ehat1.reshape(NC, E, HH))

    eec2 = _eec2(e, ehat1.reshape(NC, E, HH), stats1,
                 params['emb_e_w'], r(params['emb_e_b']),
                 r(p1['bn_e_g']), r(p1['bn_e_b']), p2['C'], r(p2['e_b']))
    ta2, ts2 = _tables(h1, p2['A'], p2['B'], p2['V'])
    [nd2] = _sc_edge_rest(
        eec2.reshape(NC * E // 2, H), ta2, ts2.reshape(NC * N, H), dst, src)
    h2 = _node_update(h1, nd2.reshape(NC, N, H), p2['U'], r(p2['h_b']),
                      r(p2['bn_h_g']), r(p2['bn_h_b']))

    w2p = jnp.pad(params['mlp_w2'], ((0, 0), (0, H - 2)))
    b2p = jnp.pad(params['mlp_b2'], (0, H - 2)).reshape(1, H)
    out = _head(h2, params['mlp_w1'], r(params['mlp_b1']), w2p, b2p)
    return out[:, :2]
